# baseline (speedup 1.0000x reference)
import jax
import jax.numpy as jnp
from jax import lax
from jax.experimental import pallas as pl
from jax.experimental.pallas import tpu as pltpu
from jax.experimental.pallas import tpu_sc as plsc

NC, NS, L = 2, 16, 16
NW = NC * NS
CHUNK = 128
B, S = 16384, 26
N_IDX = B * S
N_CHUNKS = N_IDX // CHUNK
CPW = N_CHUNKS // NW
D = 3
WORDS = CHUNK * D


def _sc_body(idx_hbm, w1_hbm, w2_hbm, out_hbm,
             idx_v, rows1_v, rows2_v, out_v, sem1, sem2):
    wid = lax.axis_index("s") * NC + lax.axis_index("c")

    i16 = lax.iota(jnp.int32, L)
    rpats = [(i16 + t * L) // 3 for t in range(3)]
    cpats = [(i16 + t * L) % 3 for t in range(3)]

    def chunk_body(j, carry):
        g = wid * CPW + j
        pltpu.sync_copy(idx_hbm.at[g], idx_v)
        c1 = pltpu.async_copy(w1_hbm.at[idx_v], rows1_v, sem1)
        c2 = pltpu.async_copy(w2_hbm.at[idx_v], rows2_v, sem2)
        c1.wait()
        c2.wait()
        for m in range(CHUNK // L):
            for t in range(3):
                r = rpats[t] + m * L
                a = plsc.load_gather(rows1_v, [r, cpats[t]])
                bv = plsc.load_gather(rows2_v, [r, cpats[t]])
                out_v[pl.ds(m * 3 * L + t * L, L)] = a + bv
        pltpu.sync_copy(out_v, out_hbm.at[pl.ds(g * WORDS, WORDS)])
        return carry

    lax.fori_loop(0, CPW, chunk_body, 0)


def kernel(inputs, W1, W2):
    idx = inputs.reshape(N_CHUNKS, CHUNK).astype(jnp.int32)
    out_flat = pl.kernel(
        _sc_body,
        out_type=jax.ShapeDtypeStruct((N_IDX * D,), jnp.float32),
        mesh=plsc.VectorSubcoreMesh(core_axis_name="c", subcore_axis_name="s"),
        compiler_params=pltpu.CompilerParams(
            use_tc_tiling_on_sc=False, needs_layout_passes=False),
        scratch_types=[
            pltpu.VMEM((CHUNK,), jnp.int32),
            pltpu.VMEM((CHUNK, D), jnp.float32),
            pltpu.VMEM((CHUNK, D), jnp.float32),
            pltpu.VMEM((WORDS,), jnp.float32),
            pltpu.SemaphoreType.DMA,
            pltpu.SemaphoreType.DMA,
        ],
    )(idx, W1, W2)
    return out_flat.reshape(B, S, D)


# speedup vs baseline: 1.0341x; 1.0341x over previous
import jax
import jax.numpy as jnp
from jax import lax
from jax.experimental import pallas as pl
from jax.experimental.pallas import tpu as pltpu
from jax.experimental.pallas import tpu_sc as plsc

NC, NS, L = 2, 16, 16
NW = NC * NS
CHUNK = 128
B, S = 16384, 26
N_IDX = B * S
D = 3
N_WORDS = N_IDX * D
N_CHUNKS = N_WORDS // CHUNK
CPW = N_CHUNKS // NW


def _sc_body(widx_hbm, w1_hbm, w2_hbm, out_hbm,
             widx_v, rows1_v, rows2_v, acc_v, sem1, sem2):
    wid = lax.axis_index("s") * NC + lax.axis_index("c")

    def chunk_body(j, carry):
        g = wid * CPW + j
        pltpu.sync_copy(widx_hbm.at[g], widx_v)
        c1 = pltpu.async_copy(w1_hbm.at[widx_v], rows1_v, sem1)
        c2 = pltpu.async_copy(w2_hbm.at[widx_v], rows2_v, sem2)
        c1.wait()
        c2.wait()
        base = j * CHUNK
        for m in range(CHUNK // L):
            sl = pl.ds(m * L, L)
            acc_v[pl.ds(base + m * L, L)] = rows1_v[sl] + rows2_v[sl]
        return carry

    lax.fori_loop(0, CPW, chunk_body, 0)
    pltpu.sync_copy(acc_v, out_hbm.at[pl.ds(wid * CPW * CHUNK, CPW * CHUNK)])


def kernel(inputs, W1, W2):
    idx_flat = inputs.reshape(-1).astype(jnp.int32)
    widx = (idx_flat[:, None] * D + jnp.arange(D, dtype=jnp.int32))
    widx = widx.reshape(N_CHUNKS, CHUNK)
    out_flat = pl.kernel(
        _sc_body,
        out_type=jax.ShapeDtypeStruct((N_WORDS,), jnp.float32),
        mesh=plsc.VectorSubcoreMesh(core_axis_name="c", subcore_axis_name="s"),
        compiler_params=pltpu.CompilerParams(
            use_tc_tiling_on_sc=False, needs_layout_passes=False),
        scratch_types=[
            pltpu.VMEM((CHUNK,), jnp.int32),
            pltpu.VMEM((CHUNK,), jnp.float32),
            pltpu.VMEM((CHUNK,), jnp.float32),
            pltpu.VMEM((CPW * CHUNK,), jnp.float32),
            pltpu.SemaphoreType.DMA,
            pltpu.SemaphoreType.DMA,
        ],
    )(widx, W1.reshape(-1), W2.reshape(-1))
    return out_flat.reshape(B, S, D)


# speedup vs baseline: 1.0789x; 1.0434x over previous
import jax
import jax.numpy as jnp
from jax import lax
from jax.experimental import pallas as pl
from jax.experimental.pallas import tpu as pltpu
from jax.experimental.pallas import tpu_sc as plsc

NC, NS, L = 2, 16, 16
NW = NC * NS
CHUNK = 128
B, S = 16384, 26
N_IDX = B * S
D = 3
N_WORDS = N_IDX * D
N_CHUNKS = N_WORDS // CHUNK
CPW = N_CHUNKS // NW
NBUF = 8
GROUPS = CPW // NBUF


def _sc_body(widx_hbm, w1_hbm, w2_hbm, out_hbm,
             widx_v, rows1_v, rows2_v, acc_v, sems):
    wid = lax.axis_index("s") * NC + lax.axis_index("c")

    pltpu.sync_copy(widx_hbm.at[pl.ds(wid * CPW, CPW)], widx_v)

    def start(jj, b):
        pltpu.async_copy(w1_hbm.at[widx_v.at[jj]], rows1_v.at[b],
                         sems.at[0, b])
        pltpu.async_copy(w2_hbm.at[widx_v.at[jj]], rows2_v.at[b],
                         sems.at[1, b])

    def drain(jj, b):
        pltpu.make_async_copy(w1_hbm.at[widx_v.at[jj]], rows1_v.at[b],
                              sems.at[0, b]).wait()
        pltpu.make_async_copy(w2_hbm.at[widx_v.at[jj]], rows2_v.at[b],
                              sems.at[1, b]).wait()

    def compute(j, b):
        base = j * CHUNK
        for m in range(CHUNK // L):
            sl = pl.ds(m * L, L)
            acc_v[pl.ds(base + m * L, L)] = (
                rows1_v.at[b][sl] + rows2_v.at[b][sl])

    for b in range(NBUF):
        start(b, b)

    def group(m, carry):
        for b in range(NBUF):
            j = m * NBUF + b
            drain(j, b)
            compute(j, b)
            start(j + NBUF, b)
        return carry

    lax.fori_loop(0, GROUPS - 1, group, 0)

    for b in range(NBUF):
        j = (GROUPS - 1) * NBUF + b
        drain(j, b)
        compute(j, b)

    pltpu.sync_copy(acc_v, out_hbm.at[pl.ds(wid * CPW * CHUNK, CPW * CHUNK)])


def kernel(inputs, W1, W2):
    idx_flat = inputs.reshape(-1).astype(jnp.int32)
    widx = (idx_flat[:, None] * D + jnp.arange(D, dtype=jnp.int32))
    widx = widx.reshape(N_CHUNKS, CHUNK)
    out_flat = pl.kernel(
        _sc_body,
        out_type=jax.ShapeDtypeStruct((N_WORDS,), jnp.float32),
        mesh=plsc.VectorSubcoreMesh(core_axis_name="c", subcore_axis_name="s"),
        compiler_params=pltpu.CompilerParams(
            use_tc_tiling_on_sc=False, needs_layout_passes=False),
        scratch_types=[
            pltpu.VMEM((CPW, CHUNK), jnp.int32),
            pltpu.VMEM((NBUF, CHUNK), jnp.float32),
            pltpu.VMEM((NBUF, CHUNK), jnp.float32),
            pltpu.VMEM((CPW * CHUNK,), jnp.float32),
            pltpu.SemaphoreType.DMA((2, NBUF)),
        ],
    )(widx, W1.reshape(-1), W2.reshape(-1))
    return out_flat.reshape(B, S, D)
